# initial kernel scaffold (unmeasured)
import jax
import jax.numpy as jnp
from jax import lax
from jax.experimental import pallas as pl
from jax.experimental.pallas import tpu as pltpu

N_DEV = 4
HALF = 512


def kernel(x, w_mat):
    m_per, k = x.shape
    _, n_per = w_mat.shape

    def body(x_ref, w_ref, out_ref, cw_ref, ccw_ref,
             cw_send, cw_recv, ccw_send, ccw_recv):
        me = lax.axis_index("i")
        right = lax.rem(me + 1, N_DEV)
        left = lax.rem(me + N_DEV - 1, N_DEV)

        def silu_gemm(a):
            y = jnp.dot(a, w_ref[:, :], preferred_element_type=jnp.float32)
            return y * jax.nn.sigmoid(y)

        barrier_sem = pltpu.get_barrier_semaphore()
        for nbr in (left, right):
            pl.semaphore_signal(
                barrier_sem, inc=1,
                device_id=(nbr,), device_id_type=pl.DeviceIdType.MESH,
            )
        pl.semaphore_wait(barrier_sem, 2)

        def make_rdma(h, direction):
            if direction == "cw":
                buf, send, recv, dst = cw_ref, cw_send, cw_recv, right
                src = x_ref.at[0:HALF] if h == 0 else buf.at[h - 1]
            else:
                buf, send, recv, dst = ccw_ref, ccw_send, ccw_recv, left
                src = x_ref.at[HALF:2 * HALF] if h == 0 else buf.at[h - 1]
            return pltpu.make_async_remote_copy(
                src_ref=src,
                dst_ref=buf.at[h],
                send_sem=send.at[h],
                recv_sem=recv.at[h],
                device_id=(dst,),
                device_id_type=pl.DeviceIdType.MESH,
            )

        rdmas = {}
        for d in ("cw", "ccw"):
            rdmas[(0, d)] = make_rdma(0, d)
            rdmas[(0, d)].start()

        out_ref[pl.ds(me * m_per, m_per), :] = silu_gemm(x_ref[:, :])

        for h in range(N_DEV - 1):
            for d in ("cw", "ccw"):
                rdmas[(h, d)].wait_recv()
                if h + 1 < N_DEV - 1:
                    rdmas[(h + 1, d)] = make_rdma(h + 1, d)
                    rdmas[(h + 1, d)].start()
            cw_origin = lax.rem(me + N_DEV - h - 1, N_DEV)
            ccw_origin = lax.rem(me + h + 1, N_DEV)
            out_ref[pl.ds(cw_origin * m_per, HALF), :] = silu_gemm(
                cw_ref[h, :, :])
            out_ref[pl.ds(ccw_origin * m_per + HALF, HALF), :] = silu_gemm(
                ccw_ref[h, :, :])

        for key in rdmas:
            rdmas[key].wait_send()

    return pl.pallas_call(
        body,
        out_shape=jax.ShapeDtypeStruct((N_DEV * m_per, n_per), jnp.float32),
        in_specs=[
            pl.BlockSpec(memory_space=pltpu.VMEM),
            pl.BlockSpec(memory_space=pltpu.VMEM),
        ],
        out_specs=pl.BlockSpec(memory_space=pltpu.VMEM),
        scratch_shapes=[
            pltpu.VMEM((N_DEV - 1, HALF, k), jnp.float32),
            pltpu.VMEM((N_DEV - 1, HALF, k), jnp.float32),
            pltpu.SemaphoreType.DMA((N_DEV - 1,)),
            pltpu.SemaphoreType.DMA((N_DEV - 1,)),
            pltpu.SemaphoreType.DMA((N_DEV - 1,)),
            pltpu.SemaphoreType.DMA((N_DEV - 1,)),
        ],
        compiler_params=pltpu.CompilerParams(collective_id=0),
    )(x, w_mat)


# baseline (device time: 300227 ns/iter reference)
import jax
import jax.numpy as jnp
from jax import lax
from jax.experimental import pallas as pl
from jax.experimental.pallas import tpu as pltpu

N_DEV = 4
SUB = 256
N_STEPS = 2 * (N_DEV - 1)
N_SLOTS = 3


def kernel(x, w_mat):
    m_per, k = x.shape
    _, n_per = w_mat.shape

    def body(x_ref, w_ref, out_ref, cw_ref, ccw_ref,
             cw_send, cw_recv, ccw_send, ccw_recv, cw_credit, ccw_credit):
        me = lax.axis_index("i")
        right = lax.rem(me + 1, N_DEV)
        left = lax.rem(me + N_DEV - 1, N_DEV)

        def silu_gemm(a):
            y = jnp.dot(a, w_ref[:, :], preferred_element_type=jnp.float32)
            return y * jax.nn.sigmoid(y)

        bufs = (cw_ref, ccw_ref)
        send_sems = (cw_send, ccw_send)
        recv_sems = (cw_recv, ccw_recv)
        credits = (cw_credit, ccw_credit)
        dst_dev = (right, left)
        upstream = (left, right)
        half_off = (0, m_per // 2)

        barrier_sem = pltpu.get_barrier_semaphore()
        for nbr in (left, right):
            pl.semaphore_signal(
                barrier_sem, inc=1,
                device_id=(nbr,), device_id_type=pl.DeviceIdType.MESH,
            )
        pl.semaphore_wait(barrier_sem, 2)

        def make_rdma(s, d):
            if s < 2:
                lo = half_off[d] + s * SUB
                src = x_ref.at[lo:lo + SUB]
            else:
                src = bufs[d].at[(s - 2) % N_SLOTS]
            return pltpu.make_async_remote_copy(
                src_ref=src,
                dst_ref=bufs[d].at[s % N_SLOTS],
                send_sem=send_sems[d].at[s % N_SLOTS],
                recv_sem=recv_sems[d].at[s % N_SLOTS],
                device_id=(dst_dev[d],),
                device_id_type=pl.DeviceIdType.MESH,
            )

        def chunk_rows(s, d):
            if d == 0:
                origin = lax.rem(me + 2 * N_DEV - 1 - s // 2, N_DEV)
            else:
                origin = lax.rem(me + 1 + s // 2, N_DEV)
            return origin * m_per + half_off[d] + (s % 2) * SUB

        rd = {}
        for s in range(N_STEPS):
            for d in (0, 1):
                if s >= 1:
                    rd[(s - 1, d)].wait_send()
                if s >= 3:
                    pl.semaphore_signal(
                        credits[d], inc=1,
                        device_id=(upstream[d],),
                        device_id_type=pl.DeviceIdType.MESH,
                    )
                    pl.semaphore_wait(credits[d], 1)
                rd[(s, d)] = make_rdma(s, d)
                rd[(s, d)].start()
            if s == 0:
                out_ref[pl.ds(me * m_per, m_per), :] = silu_gemm(x_ref[:, :])
            for d in (0, 1):
                if s >= 1:
                    rd[(s - 1, d)].wait_recv()
                    out_ref[pl.ds(chunk_rows(s - 1, d), SUB), :] = silu_gemm(
                        bufs[d][(s - 1) % N_SLOTS, :, :])

        for d in (0, 1):
            s = N_STEPS - 1
            rd[(s, d)].wait_send()
            rd[(s, d)].wait_recv()
            out_ref[pl.ds(chunk_rows(s, d), SUB), :] = silu_gemm(
                bufs[d][s % N_SLOTS, :, :])

    return pl.pallas_call(
        body,
        out_shape=jax.ShapeDtypeStruct((N_DEV * m_per, n_per), jnp.float32),
        in_specs=[
            pl.BlockSpec(memory_space=pltpu.VMEM),
            pl.BlockSpec(memory_space=pltpu.VMEM),
        ],
        out_specs=pl.BlockSpec(memory_space=pltpu.VMEM),
        scratch_shapes=[
            pltpu.VMEM((N_SLOTS, SUB, k), jnp.float32),
            pltpu.VMEM((N_SLOTS, SUB, k), jnp.float32),
            pltpu.SemaphoreType.DMA((N_SLOTS,)),
            pltpu.SemaphoreType.DMA((N_SLOTS,)),
            pltpu.SemaphoreType.DMA((N_SLOTS,)),
            pltpu.SemaphoreType.DMA((N_SLOTS,)),
            pltpu.SemaphoreType.REGULAR,
            pltpu.SemaphoreType.REGULAR,
        ],
        compiler_params=pltpu.CompilerParams(
            collective_id=0,
            vmem_limit_bytes=60 * 1024 * 1024,
        ),
    )(x, w_mat)


# device time: 294353 ns/iter; 1.0200x vs baseline; 1.0200x over previous
import jax
import jax.numpy as jnp
from jax import lax
from jax.experimental import pallas as pl
from jax.experimental.pallas import tpu as pltpu

N_DEV = 4
SUB = 128
C = 4
N_SLOTS = C + 2
N_STEPS = (N_DEV - 1) * C


def kernel(x, w_mat):
    m_per, k = x.shape
    _, n_per = w_mat.shape

    def body(x_ref, w_ref, out_ref, cw_ref, ccw_ref,
             cw_send, cw_recv, ccw_send, ccw_recv, cw_credit, ccw_credit):
        me = lax.axis_index("i")
        right = lax.rem(me + 1, N_DEV)
        left = lax.rem(me + N_DEV - 1, N_DEV)

        def silu_gemm(a):
            y = jnp.dot(a, w_ref[:, :], preferred_element_type=jnp.float32)
            return y * jax.nn.sigmoid(y)

        bufs = (cw_ref, ccw_ref)
        send_sems = (cw_send, ccw_send)
        recv_sems = (cw_recv, ccw_recv)
        credits = (cw_credit, ccw_credit)
        dst_dev = (right, left)
        upstream = (left, right)
        half_off = (0, m_per // 2)

        barrier_sem = pltpu.get_barrier_semaphore()
        for nbr in (left, right):
            pl.semaphore_signal(
                barrier_sem, inc=1,
                device_id=(nbr,), device_id_type=pl.DeviceIdType.MESH,
            )
        pl.semaphore_wait(barrier_sem, 2)

        def make_rdma(s, d):
            if s < C:
                lo = half_off[d] + s * SUB
                src = x_ref.at[lo:lo + SUB]
            else:
                src = bufs[d].at[(s - C) % N_SLOTS]
            return pltpu.make_async_remote_copy(
                src_ref=src,
                dst_ref=bufs[d].at[s % N_SLOTS],
                send_sem=send_sems[d].at[s % N_SLOTS],
                recv_sem=recv_sems[d].at[s % N_SLOTS],
                device_id=(dst_dev[d],),
                device_id_type=pl.DeviceIdType.MESH,
            )

        def chunk_rows(s, d):
            hop = s // C + 1
            if d == 0:
                origin = lax.rem(me + 2 * N_DEV - hop, N_DEV)
            else:
                origin = lax.rem(me + hop, N_DEV)
            return origin * m_per + half_off[d] + (s % C) * SUB

        rd = {}
        for d in (0, 1):
            for s in range(C):
                rd[(s, d)] = make_rdma(s, d)
                rd[(s, d)].start()

        out_ref[pl.ds(me * m_per, m_per), :] = silu_gemm(x_ref[:, :])

        for s in range(N_STEPS):
            for d in (0, 1):
                rd[(s, d)].wait_recv()
                rd[(s, d)].wait_send()
                if C <= s < C + (N_STEPS - N_SLOTS):
                    pl.semaphore_signal(
                        credits[d], inc=1,
                        device_id=(upstream[d],),
                        device_id_type=pl.DeviceIdType.MESH,
                    )
                    pl.semaphore_wait(credits[d], 1)
                    rd[(s + 2, d)] = make_rdma(s + 2, d)
                    rd[(s + 2, d)].start()
                if s < N_SLOTS - C:
                    rd[(s + C, d)] = make_rdma(s + C, d)
                    rd[(s + C, d)].start()
            for d in (0, 1):
                out_ref[pl.ds(chunk_rows(s, d), SUB), :] = silu_gemm(
                    bufs[d][s % N_SLOTS, :, :])

    return pl.pallas_call(
        body,
        out_shape=jax.ShapeDtypeStruct((N_DEV * m_per, n_per), jnp.float32),
        in_specs=[
            pl.BlockSpec(memory_space=pltpu.VMEM),
            pl.BlockSpec(memory_space=pltpu.VMEM),
        ],
        out_specs=pl.BlockSpec(memory_space=pltpu.VMEM),
        scratch_shapes=[
            pltpu.VMEM((N_SLOTS, SUB, k), jnp.float32),
            pltpu.VMEM((N_SLOTS, SUB, k), jnp.float32),
            pltpu.SemaphoreType.DMA((N_SLOTS,)),
            pltpu.SemaphoreType.DMA((N_SLOTS,)),
            pltpu.SemaphoreType.DMA((N_SLOTS,)),
            pltpu.SemaphoreType.DMA((N_SLOTS,)),
            pltpu.SemaphoreType.REGULAR,
            pltpu.SemaphoreType.REGULAR,
        ],
        compiler_params=pltpu.CompilerParams(
            collective_id=0,
            vmem_limit_bytes=60 * 1024 * 1024,
        ),
    )(x, w_mat)


# device time: 292351 ns/iter; 1.0269x vs baseline; 1.0068x over previous
import jax
import jax.numpy as jnp
from jax import lax
from jax.experimental import pallas as pl
from jax.experimental.pallas import tpu as pltpu

N_DEV = 4
SUB = 128
C = 4
N_SLOTS = C + 2
N_STEPS = (N_DEV - 1) * C


def kernel(x, w_mat):
    m_per, k = x.shape
    _, n_per = w_mat.shape

    def body(x_ref, w_ref, out_ref, cw_ref, ccw_ref,
             cw_send, cw_recv, ccw_send, ccw_recv, cw_credit, ccw_credit,
             own_stage, stage, own_osem, osems):
        me = lax.axis_index("i")
        right = lax.rem(me + 1, N_DEV)
        left = lax.rem(me + N_DEV - 1, N_DEV)

        def silu_gemm(a):
            y = jnp.dot(a, w_ref[:, :], preferred_element_type=jnp.float32)
            return y * jax.nn.sigmoid(y)

        bufs = (cw_ref, ccw_ref)
        send_sems = (cw_send, ccw_send)
        recv_sems = (cw_recv, ccw_recv)
        credits = (cw_credit, ccw_credit)
        dst_dev = (right, left)
        upstream = (left, right)
        half_off = (0, m_per // 2)

        barrier_sem = pltpu.get_barrier_semaphore()
        for nbr in (left, right):
            pl.semaphore_signal(
                barrier_sem, inc=1,
                device_id=(nbr,), device_id_type=pl.DeviceIdType.MESH,
            )
        pl.semaphore_wait(barrier_sem, 2)

        def make_rdma(s, d):
            if s < C:
                lo = half_off[d] + s * SUB
                src = x_ref.at[lo:lo + SUB]
            else:
                src = bufs[d].at[(s - C) % N_SLOTS]
            return pltpu.make_async_remote_copy(
                src_ref=src,
                dst_ref=bufs[d].at[s % N_SLOTS],
                send_sem=send_sems[d].at[s % N_SLOTS],
                recv_sem=recv_sems[d].at[s % N_SLOTS],
                device_id=(dst_dev[d],),
                device_id_type=pl.DeviceIdType.MESH,
            )

        def chunk_rows(s, d):
            hop = s // C + 1
            if d == 0:
                origin = lax.rem(me + 2 * N_DEV - hop, N_DEV)
            else:
                origin = lax.rem(me + hop, N_DEV)
            return origin * m_per + half_off[d] + (s % C) * SUB

        rd = {}
        for d in (0, 1):
            for s in range(C):
                rd[(s, d)] = make_rdma(s, d)
                rd[(s, d)].start()

        own_stage[:, :] = silu_gemm(x_ref[:, :])
        own_dma = pltpu.make_async_copy(
            own_stage, out_ref.at[pl.ds(me * m_per, m_per), :], own_osem)
        own_dma.start()

        odma = {}
        for s in range(N_STEPS):
            for d in (0, 1):
                rd[(s, d)].wait_recv()
                rd[(s, d)].wait_send()
                if C <= s < C + (N_STEPS - N_SLOTS):
                    pl.semaphore_signal(
                        credits[d], inc=1,
                        device_id=(upstream[d],),
                        device_id_type=pl.DeviceIdType.MESH,
                    )
                    pl.semaphore_wait(credits[d], 1)
                    rd[(s + 2, d)] = make_rdma(s + 2, d)
                    rd[(s + 2, d)].start()
                if s < N_SLOTS - C:
                    rd[(s + C, d)] = make_rdma(s + C, d)
                    rd[(s + C, d)].start()
            for d in (0, 1):
                if s >= 2:
                    odma[(s - 2, d)].wait()
                stage[d, s % 2, :, :] = silu_gemm(bufs[d][s % N_SLOTS, :, :])
                odma[(s, d)] = pltpu.make_async_copy(
                    stage.at[d, s % 2],
                    out_ref.at[pl.ds(chunk_rows(s, d), SUB), :],
                    osems.at[d, s % 2],
                )
                odma[(s, d)].start()

        own_dma.wait()
        for d in (0, 1):
            odma[(N_STEPS - 2, d)].wait()
            odma[(N_STEPS - 1, d)].wait()

    return pl.pallas_call(
        body,
        out_shape=jax.ShapeDtypeStruct((N_DEV * m_per, n_per), jnp.float32),
        in_specs=[
            pl.BlockSpec(memory_space=pltpu.VMEM),
            pl.BlockSpec(memory_space=pltpu.VMEM),
        ],
        out_specs=pl.BlockSpec(memory_space=pl.ANY),
        scratch_shapes=[
            pltpu.VMEM((N_SLOTS, SUB, k), jnp.float32),
            pltpu.VMEM((N_SLOTS, SUB, k), jnp.float32),
            pltpu.SemaphoreType.DMA((N_SLOTS,)),
            pltpu.SemaphoreType.DMA((N_SLOTS,)),
            pltpu.SemaphoreType.DMA((N_SLOTS,)),
            pltpu.SemaphoreType.DMA((N_SLOTS,)),
            pltpu.SemaphoreType.REGULAR,
            pltpu.SemaphoreType.REGULAR,
            pltpu.VMEM((m_per, n_per), jnp.float32),
            pltpu.VMEM((2, 2, SUB, n_per), jnp.float32),
            pltpu.SemaphoreType.DMA,
            pltpu.SemaphoreType.DMA((2, 2)),
        ],
        compiler_params=pltpu.CompilerParams(
            collective_id=0,
            vmem_limit_bytes=60 * 1024 * 1024,
        ),
    )(x, w_mat)


# device time: 292338 ns/iter; 1.0270x vs baseline; 1.0000x over previous
import jax
import jax.numpy as jnp
from jax import lax
from jax.experimental import pallas as pl
from jax.experimental.pallas import tpu as pltpu

N_DEV = 4
SUB = 128
C = 4
N_SLOTS = C + 4
N_STEPS = (N_DEV - 1) * C


def kernel(x, w_mat):
    m_per, k = x.shape
    _, n_per = w_mat.shape

    def body(x_ref, w_ref, out_ref, cw_ref, ccw_ref,
             cw_send, cw_recv, ccw_send, ccw_recv, cw_credit, ccw_credit,
             own_stage, stage, own_osem, osems):
        me = lax.axis_index("i")
        right = lax.rem(me + 1, N_DEV)
        left = lax.rem(me + N_DEV - 1, N_DEV)

        def silu_gemm(a):
            y = jnp.dot(a, w_ref[:, :], preferred_element_type=jnp.float32)
            return y * jax.nn.sigmoid(y)

        bufs = (cw_ref, ccw_ref)
        send_sems = (cw_send, ccw_send)
        recv_sems = (cw_recv, ccw_recv)
        credits = (cw_credit, ccw_credit)
        dst_dev = (right, left)
        upstream = (left, right)
        half_off = (0, m_per // 2)

        barrier_sem = pltpu.get_barrier_semaphore()
        for nbr in (left, right):
            pl.semaphore_signal(
                barrier_sem, inc=1,
                device_id=(nbr,), device_id_type=pl.DeviceIdType.MESH,
            )
        pl.semaphore_wait(barrier_sem, 2)

        def make_rdma(s, d):
            if s < C:
                lo = half_off[d] + s * SUB
                src = x_ref.at[lo:lo + SUB]
            else:
                src = bufs[d].at[(s - C) % N_SLOTS]
            return pltpu.make_async_remote_copy(
                src_ref=src,
                dst_ref=bufs[d].at[s % N_SLOTS],
                send_sem=send_sems[d].at[s % N_SLOTS],
                recv_sem=recv_sems[d].at[s % N_SLOTS],
                device_id=(dst_dev[d],),
                device_id_type=pl.DeviceIdType.MESH,
            )

        def chunk_rows(s, d):
            hop = s // C + 1
            if d == 0:
                origin = lax.rem(me + 2 * N_DEV - hop, N_DEV)
            else:
                origin = lax.rem(me + hop, N_DEV)
            return origin * m_per + half_off[d] + (s % C) * SUB

        rd = {}
        for d in (0, 1):
            for s in range(C):
                rd[(s, d)] = make_rdma(s, d)
                rd[(s, d)].start()

        own_stage[:, :] = silu_gemm(x_ref[:, :])
        own_dma = pltpu.make_async_copy(
            own_stage, out_ref.at[pl.ds(me * m_per, m_per), :], own_osem)
        own_dma.start()

        odma = {}
        for s in range(N_STEPS):
            for d in (0, 1):
                rd[(s, d)].wait_recv()
                rd[(s, d)].wait_send()
                if C <= s < C + (N_STEPS - N_SLOTS):
                    pl.semaphore_signal(
                        credits[d], inc=1,
                        device_id=(upstream[d],),
                        device_id_type=pl.DeviceIdType.MESH,
                    )
                if s < N_STEPS - C:
                    if s >= N_SLOTS - C:
                        pl.semaphore_wait(credits[d], 1)
                    rd[(s + C, d)] = make_rdma(s + C, d)
                    rd[(s + C, d)].start()
            for d in (0, 1):
                if s >= 2:
                    odma[(s - 2, d)].wait()
                stage[d, s % 2, :, :] = silu_gemm(bufs[d][s % N_SLOTS, :, :])
                odma[(s, d)] = pltpu.make_async_copy(
                    stage.at[d, s % 2],
                    out_ref.at[pl.ds(chunk_rows(s, d), SUB), :],
                    osems.at[d, s % 2],
                )
                odma[(s, d)].start()

        own_dma.wait()
        for d in (0, 1):
            odma[(N_STEPS - 2, d)].wait()
            odma[(N_STEPS - 1, d)].wait()

    return pl.pallas_call(
        body,
        out_shape=jax.ShapeDtypeStruct((N_DEV * m_per, n_per), jnp.float32),
        in_specs=[
            pl.BlockSpec(memory_space=pltpu.VMEM),
            pl.BlockSpec(memory_space=pltpu.VMEM),
        ],
        out_specs=pl.BlockSpec(memory_space=pl.ANY),
        scratch_shapes=[
            pltpu.VMEM((N_SLOTS, SUB, k), jnp.float32),
            pltpu.VMEM((N_SLOTS, SUB, k), jnp.float32),
            pltpu.SemaphoreType.DMA((N_SLOTS,)),
            pltpu.SemaphoreType.DMA((N_SLOTS,)),
            pltpu.SemaphoreType.DMA((N_SLOTS,)),
            pltpu.SemaphoreType.DMA((N_SLOTS,)),
            pltpu.SemaphoreType.REGULAR,
            pltpu.SemaphoreType.REGULAR,
            pltpu.VMEM((m_per, n_per), jnp.float32),
            pltpu.VMEM((2, 2, SUB, n_per), jnp.float32),
            pltpu.SemaphoreType.DMA,
            pltpu.SemaphoreType.DMA((2, 2)),
        ],
        compiler_params=pltpu.CompilerParams(
            collective_id=0,
            vmem_limit_bytes=62 * 1024 * 1024,
        ),
    )(x, w_mat)
